# baseline (speedup 1.0000x reference)
import functools

import jax
import jax.numpy as jnp
from jax import lax
from jax.experimental import pallas as pl
from jax.experimental.pallas import tpu as pltpu
from jax.experimental.pallas import tpu_sc as plsc

NC = 2
NS = 16
L = 16

CHUNK = 80


def _spmm_body(nchunks, n, dh,
               src_hbm, dst_hbm, val_hbm, y_hbm, b_hbm, out_hbm,
               src_v, dst_v, val_v, bufs, ibuf, bv, acc,
               gs0, gs1, gs2, gs3, ss0, ss1, ss2, ss3):
    gsems = (gs0, gs1, gs2, gs3)
    ssems = (ss0, ss1, ss2, ss3)
    c = lax.axis_index("c")
    s = lax.axis_index("s")

    pltpu.sync_copy(src_hbm.at[s], src_v)
    pltpu.sync_copy(dst_hbm.at[s], dst_v)
    pltpu.sync_copy(val_hbm.at[s], val_v)
    pltpu.sync_copy(b_hbm.at[c], bv)

    slab = (n // NS) & ~7
    rem = n - NS * slab
    nd16 = dh // L

    irows = ibuf.shape[0]
    for q in range(nd16):
        bq = bv[pl.ds(q * L, L)]
        for i in range(irows):
            ibuf[i, pl.ds(q * L, L)] = bq
    for k in range(slab // irows):
        pltpu.sync_copy(ibuf, acc.at[pl.ds(s * slab + k * irows, irows)])

    if rem:
        @pl.when(s == NS - 1)
        def _():
            pltpu.sync_copy(ibuf.at[pl.ds(0, rem)], acc.at[pl.ds(NS * slab, rem)])

    plsc.subcore_barrier()

    def gather_start(j, b):
        pltpu.async_copy(y_hbm.at[c].at[src_v.at[j]], bufs.at[b], gsems[b])

    def gather_wait(j, b):
        pltpu.make_async_copy(y_hbm.at[c].at[src_v.at[j]], bufs.at[b],
                              gsems[b]).wait()

    def scat_start(j, b):
        pltpu.async_copy(bufs.at[b], acc.at[dst_v.at[j]], ssems[b], add=True)

    def scat_wait(j, b):
        pltpu.make_async_copy(bufs.at[b], acc.at[dst_v.at[j]],
                              ssems[b]).wait()

    def compute(j, b):
        for g in range(CHUNK // L):
            vals = val_v[j, pl.ds(g * L, L)]
            for i in range(L):
                v = lax.gather(
                    vals, jnp.full((L, 1), i, jnp.int32),
                    lax.GatherDimensionNumbers(
                        offset_dims=(), collapsed_slice_dims=(0,),
                        start_index_map=(0,)),
                    (1,), mode=lax.GatherScatterMode.PROMISE_IN_BOUNDS)
                r = g * L + i
                for q in range(nd16):
                    bufs[b, r, pl.ds(q * L, L)] = (
                        bufs[b, r, pl.ds(q * L, L)] * v)

    def step(j, b, wait_scat):
        gather_wait(j, b)
        compute(j, b)
        scat_start(j, b)
        b2 = (b + 2) % 4
        if wait_scat:
            scat_wait(j - 2, b2)
        gather_start(j + 2, b2)

    gather_start(0, 0)
    gather_start(1, 1)
    step(0, 0, False)
    step(1, 1, False)
    step(2, 2, True)
    step(3, 3, True)

    def quad(i, carry):
        for b in range(4):
            step(i * 4 + b, b, True)
        return carry

    lax.fori_loop(1, (nchunks - 2) // 4, quad, 0)

    for j, b in ((nchunks - 2, 0), (nchunks - 1, 1)):
        gather_wait(j, b)
        compute(j, b)
        scat_start(j, b)

    for j in range(nchunks - 4, nchunks):
        scat_wait(j, j % 4)
    plsc.subcore_barrier()

    pltpu.sync_copy(acc.at[pl.ds(s * slab, slab)],
                    out_hbm.at[pl.ds(s * slab, slab), c])
    if rem:
        @pl.when(s == NS - 1)
        def _():
            pltpu.sync_copy(acc.at[pl.ds(NS * slab, rem)],
                            out_hbm.at[pl.ds(NS * slab, rem), c])


def _matmul_body(dh, x_ref, w_ref, y_ref):
    x = x_ref[...]
    y_ref[0] = jnp.dot(x, w_ref[:dh].T, preferred_element_type=jnp.float32)
    y_ref[1] = jnp.dot(x, w_ref[dh:].T, preferred_element_type=jnp.float32)


def kernel(edge_index, edge_values, X, W, b):
    n, d = X.shape
    d_out = W.shape[0]
    dh = d_out // NC
    e = edge_values.shape[0]
    edges_per_tile = e // NS
    nchunks = edges_per_tile // CHUNK
    assert nchunks % 4 == 2 and nchunks >= 6
    slab = (n // NS) & ~7
    irows = max(k for k in range(1, 105) if slab % k == 0)

    src = edge_index[1].astype(jnp.int32).reshape(NS, nchunks, CHUNK)
    dst = edge_index[0].astype(jnp.int32).reshape(NS, nchunks, CHUNK)
    val = edge_values.astype(jnp.float32).reshape(NS, nchunks, CHUNK)

    blk = 1000
    y = pl.pallas_call(
        functools.partial(_matmul_body, dh),
        grid=(n // blk,),
        in_specs=[
            pl.BlockSpec((blk, d), lambda i: (i, 0)),
            pl.BlockSpec((d_out, d), lambda i: (0, 0)),
        ],
        out_specs=pl.BlockSpec((NC, blk, dh), lambda i: (0, i, 0)),
        out_shape=jax.ShapeDtypeStruct((NC, n, dh), jnp.float32),
    )(X, W)

    mesh = plsc.VectorSubcoreMesh(core_axis_name="c", subcore_axis_name="s")
    spmm = pl.kernel(
        functools.partial(_spmm_body, nchunks, n, dh),
        out_type=jax.ShapeDtypeStruct((n, NC, dh), jnp.float32),
        mesh=mesh,
        compiler_params=pltpu.CompilerParams(use_tc_tiling_on_sc=False),
        scratch_types=[
            pltpu.VMEM((nchunks, CHUNK), jnp.int32),
            pltpu.VMEM((nchunks, CHUNK), jnp.int32),
            pltpu.VMEM((nchunks, CHUNK), jnp.float32),
            pltpu.VMEM((4, CHUNK, dh), jnp.float32),
            pltpu.VMEM((irows, dh), jnp.float32),
            pltpu.VMEM((dh,), jnp.float32),
            pltpu.VMEM_SHARED((n, dh), jnp.float32),
            pltpu.SemaphoreType.DMA,
            pltpu.SemaphoreType.DMA,
            pltpu.SemaphoreType.DMA,
            pltpu.SemaphoreType.DMA,
            pltpu.SemaphoreType.DMA,
            pltpu.SemaphoreType.DMA,
            pltpu.SemaphoreType.DMA,
            pltpu.SemaphoreType.DMA,
        ],
    )
    out2 = spmm(src, dst, val, y, b.reshape(NC, dh))
    return out2.reshape(n, d_out)
